# baseline (device time: 75669 ns/iter reference)
import jax
import jax.numpy as jnp
from jax import lax
from jax.experimental import pallas as pl
from jax.experimental.pallas import tpu as pltpu

N_DEV = 4
E_PER = 8
N_EXP = 32
N_TOK = 2048
D = 512
H = 1024
CAP = 640
SEG = CAP // 2


def kernel(x, router_W, route_idx, expert_W, shared_W):
    def body(x_ref, rw_ref, idx_ref, ew_ref, sw_ref, out_ref,
             xw_ref, ewb_ref, xcat_ref, y_ref, yin_ref, snd, rcv):
        my = lax.axis_index("i")
        left = lax.rem(my + N_DEV - 1, N_DEV)
        right = lax.rem(my + 1, N_DEV)
        opp = lax.rem(my + 2, N_DEV)

        barrier = pltpu.get_barrier_semaphore()
        for nbr in (left, right, opp):
            pl.semaphore_signal(barrier, inc=1, device_id=(nbr,),
                                device_id_type=pl.DeviceIdType.MESH)

        xf = x_ref[:, :]
        xb = xf.astype(jnp.bfloat16)
        scores = jnp.dot(xb, rw_ref[:, :].astype(jnp.bfloat16),
                         preferred_element_type=jnp.float32)
        probs = jnp.exp(scores - jnp.max(scores, axis=1, keepdims=True))
        probs = probs / jnp.sum(probs, axis=1, keepdims=True)
        idx_all = idx_ref[:, :]
        e_iota = lax.broadcasted_iota(jnp.int32, (N_TOK, N_EXP), 1)
        p_sel = jnp.sum(jnp.where(e_iota == idx_all, probs, 0.0),
                        axis=1, keepdims=True)
        xw_ref[:, :] = (xf * p_sel).astype(jnp.bfloat16)
        ewb_ref[:, :] = ew_ref[:, :, :].astype(jnp.bfloat16).reshape(
            E_PER * D, H)

        chip_of = lax.div(idx_all, E_PER)
        chips_iota = lax.broadcasted_iota(jnp.int32, (N_TOK, N_DEV), 1)
        masks = (chip_of == chips_iota).astype(jnp.float32)
        HB = N_TOK // 2
        tri = (lax.broadcasted_iota(jnp.int32, (HB, HB), 0)
               >= lax.broadcasted_iota(jnp.int32, (HB, HB), 1)
               ).astype(jnp.bfloat16)
        m0 = masks[0:HB, :].astype(jnp.bfloat16)
        m1 = masks[HB:N_TOK, :].astype(jnp.bfloat16)
        p0 = jnp.dot(tri, m0, preferred_element_type=jnp.float32)
        p1 = jnp.dot(tri, m1, preferred_element_type=jnp.float32)
        ranks = (jnp.concatenate([p0, p1 + p0[HB - 1:HB, :]], axis=0)
                 - masks).astype(jnp.int32)
        cap_iota = lax.broadcasted_iota(jnp.int32, (N_TOK, CAP), 1)

        def scatter_mat(r):
            sel = jnp.sum(jnp.where(chips_iota == r, ranks, 0),
                          axis=1, keepdims=True)
            sel = jnp.where(chip_of == r, sel, -1)
            sel = jnp.broadcast_to(sel, (N_TOK, CAP))
            return jnp.where(sel == cap_iota, 1, 0).astype(jnp.bfloat16)

        def tdot(a, b):
            return lax.dot_general(a, b, (((0,), (0,)), ((), ())),
                                   preferred_element_type=jnp.float32)

        S_my = scatter_mat(my)

        def compute_seg(s):
            sl = slice(s * SEG, (s + 1) * SEG)
            xg = tdot(S_my[:, sl], xw_ref[:, :]).astype(jnp.bfloat16)
            idxg = tdot(S_my[:, sl], idx_all.astype(jnp.bfloat16))
            for e in range(E_PER):
                ge = my * E_PER + e
                xcat_ref[:, e * D:(e + 1) * D] = jnp.where(
                    idxg == ge.astype(jnp.float32), xg,
                    jnp.zeros((), jnp.bfloat16))
            y_ref[sl, :] = jnp.dot(xcat_ref[:, :], ewb_ref[:, :],
                                   preferred_element_type=jnp.float32
                                   ).astype(jnp.bfloat16)

        started = []

        def send(src_rows, dst_slot, sem_idx, dev, src_buf=None):
            dd = pltpu.make_async_remote_copy(
                src_ref=(y_ref if src_buf is None else src_buf).at[src_rows],
                dst_ref=yin_ref.at[dst_slot, src_rows],
                send_sem=snd.at[sem_idx], recv_sem=rcv.at[sem_idx],
                device_id=(dev,), device_id_type=pl.DeviceIdType.MESH)
            dd.start()
            started.append(dd)

        def recv_wait(dst_slot, src_rows, sem_idx):
            pltpu.make_async_remote_copy(
                src_ref=y_ref.at[src_rows], dst_ref=yin_ref.at[dst_slot, src_rows],
                send_sem=snd.at[sem_idx], recv_sem=rcv.at[sem_idx],
                device_id=(right,), device_id_type=pl.DeviceIdType.MESH
            ).wait_recv()

        s0 = pl.ds(0, SEG)
        s1 = pl.ds(SEG, SEG)

        compute_seg(0)
        pl.semaphore_wait(barrier, 3)
        send(s0, 2, 4, opp)
        send(s0, 0, 0, right)
        send(s0, 1, 2, left)
        compute_seg(1)
        send(s1, 2, 5, opp)
        send(s1, 0, 1, right)
        send(s1, 1, 3, left)

        swb = sw_ref[:, :].astype(jnp.bfloat16)
        acc = jnp.dot(xb, swb, preferred_element_type=jnp.float32
                      ).astype(jnp.bfloat16)
        acc = acc + jnp.dot(S_my, y_ref[:, :],
                            preferred_element_type=jnp.float32
                            ).astype(jnp.bfloat16)

        recv_wait(0, s0, 0)
        recv_wait(0, s1, 1)
        acc = acc + jnp.dot(scatter_mat(left), yin_ref[0, :, :],
                            preferred_element_type=jnp.float32
                            ).astype(jnp.bfloat16)
        recv_wait(1, s0, 2)
        recv_wait(1, s1, 3)
        acc = acc + jnp.dot(scatter_mat(right), yin_ref[1, :, :],
                            preferred_element_type=jnp.float32
                            ).astype(jnp.bfloat16)
        S_opp = scatter_mat(opp)
        recv_wait(2, s0, 4)
        acc = acc + jnp.dot(S_opp[:, 0:SEG], yin_ref[2, 0:SEG, :],
                            preferred_element_type=jnp.float32
                            ).astype(jnp.bfloat16)
        recv_wait(2, s1, 5)
        acc = acc + jnp.dot(S_opp[:, SEG:CAP], yin_ref[2, SEG:CAP, :],
                            preferred_element_type=jnp.float32
                            ).astype(jnp.bfloat16)

        out_ref[:, :] = acc

        for dd in started:
            dd.wait_send()

    return pl.pallas_call(
        body,
        out_shape=jax.ShapeDtypeStruct((N_TOK, H), jnp.bfloat16),
        in_specs=[pl.BlockSpec(memory_space=pltpu.VMEM)] * 5,
        out_specs=pl.BlockSpec(memory_space=pltpu.VMEM),
        scratch_shapes=[
            pltpu.VMEM((N_TOK, D), jnp.bfloat16),
            pltpu.VMEM((E_PER * D, H), jnp.bfloat16),
            pltpu.VMEM((SEG, E_PER * D), jnp.bfloat16),
            pltpu.VMEM((CAP, H), jnp.bfloat16),
            pltpu.VMEM((N_DEV - 1, CAP, H), jnp.bfloat16),
            pltpu.SemaphoreType.DMA((6,)),
            pltpu.SemaphoreType.DMA((6,)),
        ],
        compiler_params=pltpu.CompilerParams(
            collective_id=0, vmem_limit_bytes=100 * 1024 * 1024),
    )(x, router_W, route_idx, expert_W, shared_W)


# device time: 60068 ns/iter; 1.2597x vs baseline; 1.2597x over previous
import jax
import jax.numpy as jnp
from jax import lax
from jax.experimental import pallas as pl
from jax.experimental.pallas import tpu as pltpu

N_DEV = 4
E_PER = 8
N_EXP = 32
N_TOK = 2048
D = 512
H = 1024
CAP = 640
SEGN = 4
SEG = CAP // SEGN


def kernel(x, router_W, route_idx, expert_W, shared_W):
    def body(x_ref, rw_ref, idx_ref, ew_ref, sw_ref, out_ref,
             xw_ref, ewb_ref, xcat_ref, y_ref, yin_ref, snd, rcv):
        my = lax.axis_index("i")
        left = lax.rem(my + N_DEV - 1, N_DEV)
        right = lax.rem(my + 1, N_DEV)
        opp = lax.rem(my + 2, N_DEV)

        barrier = pltpu.get_barrier_semaphore()
        for nbr in (left, right):
            pl.semaphore_signal(barrier, inc=1, device_id=(nbr,),
                                device_id_type=pl.DeviceIdType.MESH)

        xf = x_ref[:, :]
        xb = xf.astype(jnp.bfloat16)
        scores = jnp.dot(xb, rw_ref[:, :].astype(jnp.bfloat16),
                         preferred_element_type=jnp.float32)
        probs = jnp.exp(scores - jnp.max(scores, axis=1, keepdims=True))
        probs = probs / jnp.sum(probs, axis=1, keepdims=True)
        idx_all = idx_ref[:, :]
        e_iota = lax.broadcasted_iota(jnp.int32, (N_TOK, N_EXP), 1)
        p_sel = jnp.sum(jnp.where(e_iota == idx_all, probs, 0.0),
                        axis=1, keepdims=True)
        xw_ref[:, :] = (xf * p_sel).astype(jnp.bfloat16)
        ewb_ref[:, :] = ew_ref[:, :, :].astype(jnp.bfloat16).reshape(
            E_PER * D, H)

        chip_of = lax.div(idx_all, E_PER)
        chips_iota = lax.broadcasted_iota(jnp.int32, (N_TOK, N_DEV), 1)
        masks = (chip_of == chips_iota).astype(jnp.float32)
        HB = N_TOK // 2
        tri = (lax.broadcasted_iota(jnp.int32, (HB, HB), 0)
               >= lax.broadcasted_iota(jnp.int32, (HB, HB), 1)
               ).astype(jnp.bfloat16)
        m0 = masks[0:HB, :].astype(jnp.bfloat16)
        m1 = masks[HB:N_TOK, :].astype(jnp.bfloat16)
        p0 = jnp.dot(tri, m0, preferred_element_type=jnp.float32)
        p1 = jnp.dot(tri, m1, preferred_element_type=jnp.float32)
        ranks = (jnp.concatenate([p0, p1 + p0[HB - 1:HB, :]], axis=0)
                 - masks).astype(jnp.int32)
        cap_iota = lax.broadcasted_iota(jnp.int32, (N_TOK, CAP), 1)

        def scatter_mat(r):
            sel = jnp.sum(jnp.where(chips_iota == r, ranks, 0),
                          axis=1, keepdims=True)
            sel = jnp.where(chip_of == r, sel, -1)
            sel = jnp.broadcast_to(sel, (N_TOK, CAP))
            return jnp.where(sel == cap_iota, 1, 0).astype(jnp.bfloat16)

        def tdot(a, b):
            return lax.dot_general(a, b, (((0,), (0,)), ((), ())),
                                   preferred_element_type=jnp.float32)

        S_my = scatter_mat(my)

        def compute_seg(s):
            sl = slice(s * SEG, (s + 1) * SEG)
            xg = tdot(S_my[:, sl], xw_ref[:, :]).astype(jnp.bfloat16)
            idxg = tdot(S_my[:, sl], idx_all.astype(jnp.bfloat16))
            for e in range(E_PER):
                ge = my * E_PER + e
                xcat_ref[:, e * D:(e + 1) * D] = jnp.where(
                    idxg == ge.astype(jnp.float32), xg,
                    jnp.zeros((), jnp.bfloat16))
            y_ref[sl, :] = jnp.dot(xcat_ref[:, :], ewb_ref[:, :],
                                   preferred_element_type=jnp.float32
                                   ).astype(jnp.bfloat16)

        started = []

        def send(src_rows, dst_slot, sem_idx, dev, src_buf=None):
            dd = pltpu.make_async_remote_copy(
                src_ref=(y_ref if src_buf is None else src_buf).at[src_rows],
                dst_ref=yin_ref.at[dst_slot, src_rows],
                send_sem=snd.at[sem_idx], recv_sem=rcv.at[sem_idx],
                device_id=(dev,), device_id_type=pl.DeviceIdType.MESH)
            dd.start()
            started.append(dd)

        def recv_wait(dst_slot, src_rows, sem_idx):
            pltpu.make_async_remote_copy(
                src_ref=y_ref.at[src_rows],
                dst_ref=yin_ref.at[dst_slot, src_rows],
                send_sem=snd.at[sem_idx], recv_sem=rcv.at[sem_idx],
                device_id=(right,), device_id_type=pl.DeviceIdType.MESH
            ).wait_recv()

        seg_rows = [pl.ds(s * SEG, SEG) for s in range(SEGN)]

        for s in range(SEGN):
            compute_seg(s)
            if s == 0:
                pl.semaphore_wait(barrier, 2)
            send(seg_rows[s], 0, s, right)
            send(seg_rows[s], 1, 4 + s, left)

        recv_wait(0, seg_rows[0], 0)
        send(seg_rows[0], 2, 8, right, src_buf=yin_ref.at[0])

        swb = sw_ref[:, :].astype(jnp.bfloat16)
        acc = jnp.dot(xb, swb, preferred_element_type=jnp.float32
                      ).astype(jnp.bfloat16)
        acc = acc + jnp.dot(S_my, y_ref[:, :],
                            preferred_element_type=jnp.float32
                            ).astype(jnp.bfloat16)

        for s in range(1, SEGN):
            recv_wait(0, seg_rows[s], s)
            send(seg_rows[s], 2, 8 + s, right, src_buf=yin_ref.at[0])
        acc = acc + jnp.dot(scatter_mat(left), yin_ref[0, :, :],
                            preferred_element_type=jnp.float32
                            ).astype(jnp.bfloat16)
        for s in range(SEGN):
            recv_wait(1, seg_rows[s], 4 + s)
        acc = acc + jnp.dot(scatter_mat(right), yin_ref[1, :, :],
                            preferred_element_type=jnp.float32
                            ).astype(jnp.bfloat16)
        S_opp = scatter_mat(opp)
        for s in range(SEGN):
            recv_wait(2, seg_rows[s], 8 + s)
            sl = slice(s * SEG, (s + 1) * SEG)
            acc = acc + jnp.dot(S_opp[:, sl], yin_ref[2, sl, :],
                                preferred_element_type=jnp.float32
                                ).astype(jnp.bfloat16)

        out_ref[:, :] = acc

        for dd in started:
            dd.wait_send()

    return pl.pallas_call(
        body,
        out_shape=jax.ShapeDtypeStruct((N_TOK, H), jnp.bfloat16),
        in_specs=[pl.BlockSpec(memory_space=pltpu.VMEM)] * 5,
        out_specs=pl.BlockSpec(memory_space=pltpu.VMEM),
        scratch_shapes=[
            pltpu.VMEM((N_TOK, D), jnp.bfloat16),
            pltpu.VMEM((E_PER * D, H), jnp.bfloat16),
            pltpu.VMEM((SEG, E_PER * D), jnp.bfloat16),
            pltpu.VMEM((CAP, H), jnp.bfloat16),
            pltpu.VMEM((N_DEV - 1, CAP, H), jnp.bfloat16),
            pltpu.SemaphoreType.DMA((12,)),
            pltpu.SemaphoreType.DMA((12,)),
        ],
        compiler_params=pltpu.CompilerParams(
            collective_id=0, vmem_limit_bytes=100 * 1024 * 1024),
    )(x, router_W, route_idx, expert_W, shared_W)


# device time: 58613 ns/iter; 1.2910x vs baseline; 1.0248x over previous
import jax
import jax.numpy as jnp
from jax import lax
from jax.experimental import pallas as pl
from jax.experimental.pallas import tpu as pltpu

N_DEV = 4
E_PER = 8
N_EXP = 32
N_TOK = 2048
D = 512
H = 1024
CAP = 640
SEGN = 4
SEG = CAP // SEGN


def kernel(x, router_W, route_idx, expert_W, shared_W):
    def body(x_ref, rw_ref, idx_ref, ew_ref, sw_ref, out_ref,
             xw_ref, ewb_ref, xcat_ref, y_ref, yin_ref, snd, rcv):
        my = lax.axis_index("i")
        left = lax.rem(my + N_DEV - 1, N_DEV)
        right = lax.rem(my + 1, N_DEV)
        opp = lax.rem(my + 2, N_DEV)

        barrier = pltpu.get_barrier_semaphore()
        for nbr in (left, right):
            pl.semaphore_signal(barrier, inc=1, device_id=(nbr,),
                                device_id_type=pl.DeviceIdType.MESH)

        xf = x_ref[:, :]
        xb = xf.astype(jnp.bfloat16)
        scores = jnp.dot(xb, rw_ref[:, :].astype(jnp.bfloat16),
                         preferred_element_type=jnp.float32)
        probs = jnp.exp(scores - jnp.max(scores, axis=1, keepdims=True))
        probs = probs / jnp.sum(probs, axis=1, keepdims=True)
        idx_all = idx_ref[:, :]
        e_iota = lax.broadcasted_iota(jnp.int32, (N_TOK, N_EXP), 1)
        p_sel = jnp.sum(jnp.where(e_iota == idx_all, probs, 0.0),
                        axis=1, keepdims=True)
        xw_ref[:, :] = (xf * p_sel).astype(jnp.bfloat16)
        ewb_ref[:, :] = ew_ref[:, :, :].astype(jnp.bfloat16).reshape(
            E_PER * D, H)

        chip_of = lax.div(idx_all, E_PER)
        chips_iota = lax.broadcasted_iota(jnp.int32, (N_TOK, N_DEV), 1)
        masks = (chip_of == chips_iota).astype(jnp.float32)
        HB = N_TOK // 2
        tri = (lax.broadcasted_iota(jnp.int32, (HB, HB), 0)
               >= lax.broadcasted_iota(jnp.int32, (HB, HB), 1)
               ).astype(jnp.bfloat16)
        m0 = masks[0:HB, :].astype(jnp.bfloat16)
        m1 = masks[HB:N_TOK, :].astype(jnp.bfloat16)
        p0 = jnp.dot(tri, m0, preferred_element_type=jnp.float32)
        p1 = jnp.dot(tri, m1, preferred_element_type=jnp.float32)
        ranks = (jnp.concatenate([p0, p1 + p0[HB - 1:HB, :]], axis=0)
                 - masks).astype(jnp.int32)
        cap_iota = lax.broadcasted_iota(jnp.int32, (N_TOK, CAP), 1)

        def scatter_mat(r):
            sel = jnp.sum(jnp.where(chips_iota == r, ranks, 0),
                          axis=1, keepdims=True)
            sel = jnp.where(chip_of == r, sel, -1)
            sel = jnp.broadcast_to(sel, (N_TOK, CAP))
            return jnp.where(sel == cap_iota, 1, 0).astype(jnp.bfloat16)

        def tdot(a, b):
            return lax.dot_general(a, b, (((0,), (0,)), ((), ())),
                                   preferred_element_type=jnp.float32)

        S_my = scatter_mat(my)

        def compute_seg(s):
            sl = slice(s * SEG, (s + 1) * SEG)
            xg = tdot(S_my[:, sl], xw_ref[:, :]).astype(jnp.bfloat16)
            idxg = tdot(S_my[:, sl], idx_all.astype(jnp.bfloat16))
            for e in range(E_PER):
                ge = my * E_PER + e
                xcat_ref[:, e * D:(e + 1) * D] = jnp.where(
                    idxg == ge.astype(jnp.float32), xg,
                    jnp.zeros((), jnp.bfloat16))
            y_ref[sl, :] = jnp.dot(xcat_ref[:, :], ewb_ref[:, :],
                                   preferred_element_type=jnp.float32
                                   ).astype(jnp.bfloat16)

        started = []

        def send(src_rows, dst_slot, sem_idx, dev, src_buf=None):
            dd = pltpu.make_async_remote_copy(
                src_ref=(y_ref if src_buf is None else src_buf).at[src_rows],
                dst_ref=yin_ref.at[dst_slot, src_rows],
                send_sem=snd.at[sem_idx], recv_sem=rcv.at[sem_idx],
                device_id=(dev,), device_id_type=pl.DeviceIdType.MESH)
            dd.start()
            started.append(dd)

        def recv_wait(dst_slot, src_rows, sem_idx):
            pltpu.make_async_remote_copy(
                src_ref=y_ref.at[src_rows],
                dst_ref=yin_ref.at[dst_slot, src_rows],
                send_sem=snd.at[sem_idx], recv_sem=rcv.at[sem_idx],
                device_id=(right,), device_id_type=pl.DeviceIdType.MESH
            ).wait_recv()

        seg_rows = [pl.ds(s * SEG, SEG) for s in range(SEGN)]

        for s in range(SEGN):
            compute_seg(s)
            if s == 0:
                pl.semaphore_wait(barrier, 2)
            send(seg_rows[s], 0, s, right)
            send(seg_rows[s], 1, 4 + s, left)

        recv_wait(0, seg_rows[0], 0)
        send(seg_rows[0], 2, 8, right, src_buf=yin_ref.at[0])
        recv_wait(0, seg_rows[1], 1)
        send(seg_rows[1], 2, 9, right, src_buf=yin_ref.at[0])

        swb = sw_ref[:, :].astype(jnp.bfloat16)
        acc = jnp.dot(xb, swb, preferred_element_type=jnp.float32
                      ).astype(jnp.bfloat16)
        acc = acc + jnp.dot(S_my, y_ref[:, :],
                            preferred_element_type=jnp.float32
                            ).astype(jnp.bfloat16)

        recv_wait(1, seg_rows[2], 6)
        send(seg_rows[2], 2, 10, left, src_buf=yin_ref.at[1])
        recv_wait(1, seg_rows[3], 7)
        send(seg_rows[3], 2, 11, left, src_buf=yin_ref.at[1])

        recv_wait(0, seg_rows[2], 2)
        recv_wait(0, seg_rows[3], 3)
        acc = acc + jnp.dot(scatter_mat(left), yin_ref[0, :, :],
                            preferred_element_type=jnp.float32
                            ).astype(jnp.bfloat16)
        recv_wait(1, seg_rows[0], 4)
        recv_wait(1, seg_rows[1], 5)
        acc = acc + jnp.dot(scatter_mat(right), yin_ref[1, :, :],
                            preferred_element_type=jnp.float32
                            ).astype(jnp.bfloat16)
        S_opp = scatter_mat(opp)
        for s in range(SEGN):
            recv_wait(2, seg_rows[s], 8 + s)
            sl = slice(s * SEG, (s + 1) * SEG)
            acc = acc + jnp.dot(S_opp[:, sl], yin_ref[2, sl, :],
                                preferred_element_type=jnp.float32
                                ).astype(jnp.bfloat16)

        out_ref[:, :] = acc

        for dd in started:
            dd.wait_send()

    return pl.pallas_call(
        body,
        out_shape=jax.ShapeDtypeStruct((N_TOK, H), jnp.bfloat16),
        in_specs=[pl.BlockSpec(memory_space=pltpu.VMEM)] * 5,
        out_specs=pl.BlockSpec(memory_space=pltpu.VMEM),
        scratch_shapes=[
            pltpu.VMEM((N_TOK, D), jnp.bfloat16),
            pltpu.VMEM((E_PER * D, H), jnp.bfloat16),
            pltpu.VMEM((SEG, E_PER * D), jnp.bfloat16),
            pltpu.VMEM((CAP, H), jnp.bfloat16),
            pltpu.VMEM((N_DEV - 1, CAP, H), jnp.bfloat16),
            pltpu.SemaphoreType.DMA((12,)),
            pltpu.SemaphoreType.DMA((12,)),
        ],
        compiler_params=pltpu.CompilerParams(
            collective_id=0, vmem_limit_bytes=100 * 1024 * 1024),
    )(x, router_W, route_idx, expert_W, shared_W)
